# Initial kernel scaffold; baseline (speedup 1.0000x reference)
import functools

import jax
import jax.numpy as jnp
from jax import lax
from jax.experimental import pallas as pl
from jax.experimental.pallas import tpu as pltpu
from jax.experimental.pallas import tpu_sc as plsc

N = 10000
E = 320000
D = 128
NC = 2
NS = 16
NW = NC * NS
C = 128
CH_W = 80
E_PAD = NW * CH_W * C
N_PAD = 10016
ROWS_T = N_PAD // NS
DEG_W = 16
BM = 400


def _sc_mesh():
    return plsc.VectorSubcoreMesh(core_axis_name="c", subcore_axis_name="s")


def _deg_body(cols_hbm, ones_hbm, zdeg_hbm, deg0, deg1, idx_v, ones_v, deg_sh):
    c = lax.axis_index("c")
    s = lax.axis_index("s")
    w = c * NS + s
    pltpu.sync_copy(zdeg_hbm, deg_sh.at[pl.ds(s * ROWS_T, ROWS_T)])
    pltpu.sync_copy(ones_hbm, ones_v)
    pltpu.sync_copy(cols_hbm.at[w], idx_v)
    plsc.subcore_barrier()

    def chunk(j, carry):
        pltpu.sync_copy(ones_v, deg_sh.at[idx_v.at[j]], add=True)
        return carry

    lax.fori_loop(0, CH_W, chunk, 0)
    plsc.subcore_barrier()

    sl = pl.ds(s * ROWS_T, ROWS_T)

    @pl.when(c == 0)
    def _():
        pltpu.sync_copy(deg_sh.at[sl], deg0.at[sl])

    @pl.when(c == 1)
    def _():
        pltpu.sync_copy(deg_sh.at[sl], deg1.at[sl])


def _deg_kernel(cols_r, ones, zdeg):
    f = pl.kernel(
        _deg_body,
        out_type=(
            jax.ShapeDtypeStruct((N_PAD, DEG_W), jnp.float32),
            jax.ShapeDtypeStruct((N_PAD, DEG_W), jnp.float32),
        ),
        mesh=_sc_mesh(),
        scratch_types=(
            pltpu.VMEM((CH_W, C), jnp.int32),
            pltpu.VMEM((C, DEG_W), jnp.float32),
            pltpu.VMEM_SHARED((N_PAD, DEG_W), jnp.float32),
        ),
    )
    return f(cols_r, ones, zdeg)


def _mm_body(x_ref, w_ref, d0_ref, d1_ref, y_ref):
    xw = jnp.dot(x_ref[...], w_ref[...], preferred_element_type=jnp.float32)
    deg = d0_ref[...][:, 0:1] + d1_ref[...][:, 0:1] + 1.0
    y_ref[...] = xw * lax.rsqrt(deg)


def _mm(x, W, deg0, deg1):
    return pl.pallas_call(
        _mm_body,
        grid=(N // BM,),
        in_specs=[
            pl.BlockSpec((BM, D), lambda i: (i, 0)),
            pl.BlockSpec((D, D), lambda i: (0, 0)),
            pl.BlockSpec((BM, DEG_W), lambda i: (i, 0)),
            pl.BlockSpec((BM, DEG_W), lambda i: (i, 0)),
        ],
        out_specs=pl.BlockSpec((BM, D), lambda i: (i, 0)),
        out_shape=jax.ShapeDtypeStruct((N, D), jnp.float32),
    )(x, W, deg0, deg1)


def _acc_body(y_hbm, rows_hbm, cols_hbm, zeros_hbm, acc0, acc1,
              rows_v, cols_v, buf, acc_sh):
    c = lax.axis_index("c")
    s = lax.axis_index("s")
    w = c * NS + s
    pltpu.sync_copy(zeros_hbm, acc_sh.at[pl.ds(s * ROWS_T, ROWS_T)])
    pltpu.sync_copy(rows_hbm.at[w], rows_v)
    pltpu.sync_copy(cols_hbm.at[w], cols_v)
    plsc.subcore_barrier()

    def step(j, carry):
        pltpu.sync_copy(y_hbm.at[rows_v.at[j]], buf)
        pltpu.sync_copy(buf, acc_sh.at[cols_v.at[j]], add=True)
        return carry

    lax.fori_loop(0, CH_W, step, 0)
    plsc.subcore_barrier()

    sl = pl.ds(s * ROWS_T, ROWS_T)

    @pl.when(c == 0)
    def _():
        pltpu.sync_copy(acc_sh.at[sl], acc0.at[sl])

    @pl.when(c == 1)
    def _():
        pltpu.sync_copy(acc_sh.at[sl], acc1.at[sl])


def _acc_kernel(y, rows_r, cols_r, zeros):
    f = pl.kernel(
        _acc_body,
        out_type=(
            jax.ShapeDtypeStruct((N_PAD, D), jnp.float32),
            jax.ShapeDtypeStruct((N_PAD, D), jnp.float32),
        ),
        mesh=_sc_mesh(),
        scratch_types=(
            pltpu.VMEM((CH_W, C), jnp.int32),
            pltpu.VMEM((CH_W, C), jnp.int32),
            pltpu.VMEM((C, D), jnp.float32),
            pltpu.VMEM_SHARED((N_PAD, D), jnp.float32),
        ),
    )
    return f(y, rows_r, cols_r, zeros)


def _fin_body(a0_ref, a1_ref, y_ref, d0_ref, d1_ref, b_ref, o_ref):
    deg = d0_ref[...][:, 0:1] + d1_ref[...][:, 0:1] + 1.0
    o_ref[...] = lax.rsqrt(deg) * (a0_ref[...] + a1_ref[...] + y_ref[...]) + b_ref[...]


def _combine(acc0, acc1, y, deg0, deg1, b2d):
    return pl.pallas_call(
        _fin_body,
        grid=(N // BM,),
        in_specs=[
            pl.BlockSpec((BM, D), lambda i: (i, 0)),
            pl.BlockSpec((BM, D), lambda i: (i, 0)),
            pl.BlockSpec((BM, D), lambda i: (i, 0)),
            pl.BlockSpec((BM, DEG_W), lambda i: (i, 0)),
            pl.BlockSpec((BM, DEG_W), lambda i: (i, 0)),
            pl.BlockSpec((1, D), lambda i: (0, 0)),
        ],
        out_specs=pl.BlockSpec((BM, D), lambda i: (i, 0)),
        out_shape=jax.ShapeDtypeStruct((N, D), jnp.float32),
    )(acc0, acc1, y, deg0, deg1, b2d)


def kernel(mpg_ft, edge_index, W, b):
    ei = edge_index.astype(jnp.int32)
    pad = E_PAD - E
    rows_p = jnp.concatenate([ei[0], jnp.zeros((pad,), jnp.int32)])
    cols_p = jnp.concatenate([ei[1], jnp.full((pad,), N, jnp.int32)])
    rows_r = rows_p.reshape(NW, CH_W, C)
    cols_r = cols_p.reshape(NW, CH_W, C)

    ones = jnp.ones((C, DEG_W), jnp.float32)
    zdeg = jnp.zeros((ROWS_T, DEG_W), jnp.float32)
    zeros = jnp.zeros((ROWS_T, D), jnp.float32)

    deg0, deg1 = _deg_kernel(cols_r, ones, zdeg)
    y = _mm(mpg_ft, W, deg0, deg1)
    acc0, acc1 = _acc_kernel(y, rows_r, cols_r, zeros)
    return _combine(acc0, acc1, y, deg0, deg1, b.reshape(1, D))


# speedup vs baseline: 13.3553x; 13.3553x over previous
import jax
import jax.numpy as jnp
from jax import lax
from jax.experimental import pallas as pl
from jax.experimental.pallas import tpu as pltpu
from jax.experimental.pallas import tpu_sc as plsc

N = 10000
E = 320000
D = 128
NC = 2
NS = 16
NW = NC * NS
C = 128
CH_W = 80
E_PAD = NW * CH_W * C
EW = CH_W * C
N_PAD = 10112
ROWS_T = N_PAD // NS
HR = 128
HRU = 80
BM = 400


def _sc_mesh():
    return plsc.VectorSubcoreMesh(core_axis_name="c", subcore_axis_name="s")


def _deg_body(cols_hbm, iota_hbm, zeros_hbm, deg_out,
              idx_v, iota_v, hist, deg_sh):
    c = lax.axis_index("c")
    s = lax.axis_index("s")
    w = c * NS + s
    pltpu.sync_copy(zeros_hbm.at[pl.ds(0, HR // NS)],
                    deg_sh.at[pl.ds(s * (HR // NS), HR // NS)])
    pltpu.sync_copy(zeros_hbm, hist)
    pltpu.sync_copy(iota_hbm, iota_v)
    pltpu.sync_copy(cols_hbm.at[w], idx_v)
    plsc.subcore_barrier()

    ones16 = jnp.ones((16,), jnp.float32)

    def step(i, carry):
        v = idx_v[pl.ds(i * 16, 16)]
        plsc.addupdate_scatter(hist, [v >> 7, v & 127], ones16)
        return carry

    lax.fori_loop(0, EW // 16, step, 0)
    pltpu.sync_copy(hist.at[pl.ds(0, HRU)], deg_sh.at[iota_v], add=True)
    plsc.subcore_barrier()

    rt = HR // NS
    pltpu.sync_copy(deg_sh.at[pl.ds(s * rt, rt)],
                    deg_out.at[c, pl.ds(s * rt, rt)])


def _deg_kernel(cols_r, iota, zeros128):
    f = pl.kernel(
        _deg_body,
        out_type=jax.ShapeDtypeStruct((NC, HR, 128), jnp.float32),
        mesh=_sc_mesh(),
        scratch_types=(
            pltpu.VMEM((EW,), jnp.int32),
            pltpu.VMEM((HRU,), jnp.int32),
            pltpu.VMEM((HR, 128), jnp.float32),
            pltpu.VMEM_SHARED((HR, 128), jnp.float32),
        ),
        compiler_params=pltpu.CompilerParams(needs_layout_passes=False),
    )
    return f(cols_r, iota, zeros128)


def _mm_body(x_ref, w_ref, d0_ref, d1_ref, y_ref):
    xw = jnp.dot(x_ref[...], w_ref[...], preferred_element_type=jnp.float32)
    deg = d0_ref[...] + d1_ref[...] + 1.0
    y_ref[...] = xw * lax.rsqrt(deg)


def _mm(x, W, deg0, deg1):
    return pl.pallas_call(
        _mm_body,
        grid=(N // BM,),
        in_specs=[
            pl.BlockSpec((BM, D), lambda i: (i, 0)),
            pl.BlockSpec((D, D), lambda i: (0, 0)),
            pl.BlockSpec((BM, 1), lambda i: (i, 0)),
            pl.BlockSpec((BM, 1), lambda i: (i, 0)),
        ],
        out_specs=pl.BlockSpec((BM, D), lambda i: (i, 0)),
        out_shape=jax.ShapeDtypeStruct((N, D), jnp.float32),
    )(x, W, deg0, deg1)


def _acc_body(y_hbm, rows_hbm, cols_hbm, zeros_hbm, acc_out,
              rows_v, cols_v, buf, acc_sh):
    c = lax.axis_index("c")
    s = lax.axis_index("s")
    w = c * NS + s
    pltpu.sync_copy(zeros_hbm, acc_sh.at[pl.ds(s * ROWS_T, ROWS_T)])
    pltpu.sync_copy(rows_hbm.at[w], rows_v)
    pltpu.sync_copy(cols_hbm.at[w], cols_v)
    plsc.subcore_barrier()

    def step(j, carry):
        pltpu.sync_copy(y_hbm.at[rows_v.at[j]], buf)
        pltpu.sync_copy(buf, acc_sh.at[cols_v.at[j]], add=True)
        return carry

    lax.fori_loop(0, CH_W, step, 0)
    plsc.subcore_barrier()

    sl = pl.ds(s * ROWS_T, ROWS_T)
    pltpu.sync_copy(acc_sh.at[sl], acc_out.at[c, sl])


def _acc_kernel(y, rows_r, cols_r, zeros):
    f = pl.kernel(
        _acc_body,
        out_type=jax.ShapeDtypeStruct((NC, N_PAD, D), jnp.float32),
        mesh=_sc_mesh(),
        scratch_types=(
            pltpu.VMEM((CH_W, C), jnp.int32),
            pltpu.VMEM((CH_W, C), jnp.int32),
            pltpu.VMEM((C, D), jnp.float32),
            pltpu.VMEM_SHARED((N_PAD, D), jnp.float32),
        ),
    )
    return f(y, rows_r, cols_r, zeros)


def _fin_body(a0_ref, a1_ref, y_ref, d0_ref, d1_ref, b_ref, o_ref):
    deg = d0_ref[...] + d1_ref[...] + 1.0
    o_ref[...] = lax.rsqrt(deg) * (a0_ref[...] + a1_ref[...] + y_ref[...]) + b_ref[...]


def _combine(acc0, acc1, y, deg0, deg1, b2d):
    return pl.pallas_call(
        _fin_body,
        grid=(N // BM,),
        in_specs=[
            pl.BlockSpec((BM, D), lambda i: (i, 0)),
            pl.BlockSpec((BM, D), lambda i: (i, 0)),
            pl.BlockSpec((BM, D), lambda i: (i, 0)),
            pl.BlockSpec((BM, 1), lambda i: (i, 0)),
            pl.BlockSpec((BM, 1), lambda i: (i, 0)),
            pl.BlockSpec((1, D), lambda i: (0, 0)),
        ],
        out_specs=pl.BlockSpec((BM, D), lambda i: (i, 0)),
        out_shape=jax.ShapeDtypeStruct((N, D), jnp.float32),
    )(acc0, acc1, y, deg0, deg1, b2d)


def kernel(mpg_ft, edge_index, W, b):
    ei = edge_index.astype(jnp.int32)
    pad = E_PAD - E
    rows_p = jnp.concatenate([ei[0], jnp.zeros((pad,), jnp.int32)])
    cols_p = jnp.concatenate([ei[1], jnp.full((pad,), N, jnp.int32)])
    rows_r = rows_p.reshape(NW, CH_W, C)
    cols_r = cols_p.reshape(NW, CH_W, C)

    iota = jnp.arange(HRU, dtype=jnp.int32)
    zeros128 = jnp.zeros((HR, 128), jnp.float32)
    zeros = jnp.zeros((ROWS_T, D), jnp.float32)

    degs = _deg_kernel(cols_p.reshape(NW, EW), iota, zeros128)
    deg0 = degs[0].reshape(HR * 128, 1)[:N]
    deg1 = degs[1].reshape(HR * 128, 1)[:N]
    y = _mm(mpg_ft, W, deg0, deg1)
    accs = _acc_kernel(y, rows_r, cols_r, zeros)
    return _combine(accs[0, :N], accs[1, :N], y, deg0, deg1, b.reshape(1, D))


# speedup vs baseline: 15.2584x; 1.1425x over previous
import jax
import jax.numpy as jnp
from jax import lax
from jax.experimental import pallas as pl
from jax.experimental.pallas import tpu as pltpu
from jax.experimental.pallas import tpu_sc as plsc

N = 10000
E = 320000
D = 128
NC = 2
NS = 16
NW = NC * NS
C = 128
CH_W = 80
E_PAD = NW * CH_W * C
EW = CH_W * C
N_PAD = 10112
ROWS_T = N_PAD // NS
HR = 128
HRU = 80
BM = 400


def _sc_mesh():
    return plsc.VectorSubcoreMesh(core_axis_name="c", subcore_axis_name="s")


def _deg_body(cols_hbm, iota_hbm, zeros_hbm, deg_out,
              idx_v, iota_v, hist, deg_sh):
    c = lax.axis_index("c")
    s = lax.axis_index("s")
    w = c * NS + s
    pltpu.sync_copy(zeros_hbm.at[pl.ds(0, HR // NS)],
                    deg_sh.at[pl.ds(s * (HR // NS), HR // NS)])
    pltpu.sync_copy(zeros_hbm, hist)
    pltpu.sync_copy(iota_hbm, iota_v)
    pltpu.sync_copy(cols_hbm.at[w], idx_v)
    plsc.subcore_barrier()

    ones16 = jnp.ones((16,), jnp.float32)

    def step(i, carry):
        v = idx_v[pl.ds(i * 16, 16)]
        plsc.addupdate_scatter(hist, [v >> 7, v & 127], ones16)
        return carry

    lax.fori_loop(0, EW // 16, step, 0)
    pltpu.sync_copy(hist.at[pl.ds(0, HRU)], deg_sh.at[iota_v], add=True)
    plsc.subcore_barrier()

    rt = HR // NS
    pltpu.sync_copy(deg_sh.at[pl.ds(s * rt, rt)],
                    deg_out.at[c, pl.ds(s * rt, rt)])


def _deg_kernel(cols_r, iota, zeros128):
    f = pl.kernel(
        _deg_body,
        out_type=jax.ShapeDtypeStruct((NC, HR, 128), jnp.float32),
        mesh=_sc_mesh(),
        scratch_types=(
            pltpu.VMEM((EW,), jnp.int32),
            pltpu.VMEM((HRU,), jnp.int32),
            pltpu.VMEM((HR, 128), jnp.float32),
            pltpu.VMEM_SHARED((HR, 128), jnp.float32),
        ),
        compiler_params=pltpu.CompilerParams(needs_layout_passes=False),
    )
    return f(cols_r, iota, zeros128)


def _mm_body(x_ref, w_ref, d0_ref, d1_ref, y_ref):
    xw = jnp.dot(x_ref[...], w_ref[...], preferred_element_type=jnp.float32)
    deg = d0_ref[...] + d1_ref[...] + 1.0
    y_ref[...] = xw * lax.rsqrt(deg)


def _mm(x, W, deg0, deg1):
    return pl.pallas_call(
        _mm_body,
        grid=(N // BM,),
        in_specs=[
            pl.BlockSpec((BM, D), lambda i: (i, 0)),
            pl.BlockSpec((D, D), lambda i: (0, 0)),
            pl.BlockSpec((BM, 1), lambda i: (i, 0)),
            pl.BlockSpec((BM, 1), lambda i: (i, 0)),
        ],
        out_specs=pl.BlockSpec((BM, D), lambda i: (i, 0)),
        out_shape=jax.ShapeDtypeStruct((N, D), jnp.float32),
    )(x, W, deg0, deg1)


def _acc_body(y_hbm, rows_hbm, cols_hbm, zeros_hbm, acc_out,
              rv0, rv1, cols_v, buf0, buf1, g0, g1, i0, i1, acc_sh):
    c = lax.axis_index("c")
    s = lax.axis_index("s")
    w = c * NS + s
    pltpu.sync_copy(zeros_hbm, acc_sh.at[pl.ds(s * ROWS_T, ROWS_T)])
    pltpu.sync_copy(cols_hbm.at[w], cols_v)
    plsc.subcore_barrier()

    pltpu.async_copy(rows_hbm.at[w, 0], rv0, i0)
    pltpu.async_copy(rows_hbm.at[w, 1], rv1, i1)
    pltpu.make_async_copy(rows_hbm.at[w, 0], rv0, i0).wait()
    pltpu.async_copy(y_hbm.at[rv0], buf0, g0)
    pltpu.make_async_copy(rows_hbm.at[w, 1], rv1, i1).wait()
    pltpu.async_copy(y_hbm.at[rv1], buf1, g1)

    def step(t, carry):
        j0 = 2 * t
        j1 = j0 + 1
        pltpu.make_async_copy(y_hbm.at[rv0], buf0, g0).wait()

        @pl.when(j0 + 2 < CH_W)
        def _():
            pltpu.async_copy(rows_hbm.at[w, j0 + 2], rv0, i0)

        pltpu.sync_copy(buf0, acc_sh.at[cols_v.at[j0]], add=True)

        @pl.when(j0 + 2 < CH_W)
        def _():
            pltpu.make_async_copy(rows_hbm.at[w, j0 + 2], rv0, i0).wait()
            pltpu.async_copy(y_hbm.at[rv0], buf0, g0)

        pltpu.make_async_copy(y_hbm.at[rv1], buf1, g1).wait()

        @pl.when(j1 + 2 < CH_W)
        def _():
            pltpu.async_copy(rows_hbm.at[w, j1 + 2], rv1, i1)

        pltpu.sync_copy(buf1, acc_sh.at[cols_v.at[j1]], add=True)

        @pl.when(j1 + 2 < CH_W)
        def _():
            pltpu.make_async_copy(rows_hbm.at[w, j1 + 2], rv1, i1).wait()
            pltpu.async_copy(y_hbm.at[rv1], buf1, g1)

        return carry

    lax.fori_loop(0, CH_W // 2, step, 0)
    plsc.subcore_barrier()

    sl = pl.ds(s * ROWS_T, ROWS_T)
    pltpu.sync_copy(acc_sh.at[sl], acc_out.at[c, sl])


def _acc_kernel(y, rows_r, cols_r, zeros):
    f = pl.kernel(
        _acc_body,
        out_type=jax.ShapeDtypeStruct((NC, N_PAD, D), jnp.float32),
        mesh=_sc_mesh(),
        scratch_types=(
            pltpu.VMEM((C,), jnp.int32),
            pltpu.VMEM((C,), jnp.int32),
            pltpu.VMEM((CH_W, C), jnp.int32),
            pltpu.VMEM((C, D), jnp.float32),
            pltpu.VMEM((C, D), jnp.float32),
            pltpu.SemaphoreType.DMA,
            pltpu.SemaphoreType.DMA,
            pltpu.SemaphoreType.DMA,
            pltpu.SemaphoreType.DMA,
            pltpu.VMEM_SHARED((N_PAD, D), jnp.float32),
        ),
    )
    return f(y, rows_r, cols_r, zeros)


def _fin_body(a0_ref, a1_ref, y_ref, d0_ref, d1_ref, b_ref, o_ref):
    deg = d0_ref[...] + d1_ref[...] + 1.0
    o_ref[...] = lax.rsqrt(deg) * (a0_ref[...] + a1_ref[...] + y_ref[...]) + b_ref[...]


def _combine(acc0, acc1, y, deg0, deg1, b2d):
    return pl.pallas_call(
        _fin_body,
        grid=(N // BM,),
        in_specs=[
            pl.BlockSpec((BM, D), lambda i: (i, 0)),
            pl.BlockSpec((BM, D), lambda i: (i, 0)),
            pl.BlockSpec((BM, D), lambda i: (i, 0)),
            pl.BlockSpec((BM, 1), lambda i: (i, 0)),
            pl.BlockSpec((BM, 1), lambda i: (i, 0)),
            pl.BlockSpec((1, D), lambda i: (0, 0)),
        ],
        out_specs=pl.BlockSpec((BM, D), lambda i: (i, 0)),
        out_shape=jax.ShapeDtypeStruct((N, D), jnp.float32),
    )(acc0, acc1, y, deg0, deg1, b2d)


def kernel(mpg_ft, edge_index, W, b):
    ei = edge_index.astype(jnp.int32)
    pad = E_PAD - E
    rows_p = jnp.concatenate([ei[0], jnp.zeros((pad,), jnp.int32)])
    cols_p = jnp.concatenate(
        [ei[1], N + (jnp.arange(pad, dtype=jnp.int32) % (N_PAD - N))])
    rows_r = rows_p.reshape(NW, CH_W, C)
    cols_r = cols_p.reshape(NW, CH_W, C)

    iota = jnp.arange(HRU, dtype=jnp.int32)
    zeros128 = jnp.zeros((HR, 128), jnp.float32)
    zeros = jnp.zeros((ROWS_T, D), jnp.float32)

    degs = _deg_kernel(cols_p.reshape(NW, EW), iota, zeros128)
    deg0 = degs[0].reshape(HR * 128, 1)[:N]
    deg1 = degs[1].reshape(HR * 128, 1)[:N]
    y = _mm(mpg_ft, W, deg0, deg1)
    accs = _acc_kernel(y, rows_r, cols_r, zeros)
    return _combine(accs[0, :N], accs[1, :N], y, deg0, deg1, b.reshape(1, D))


# speedup vs baseline: 16.5562x; 1.0851x over previous
import jax
import jax.numpy as jnp
from jax import lax
from jax.experimental import pallas as pl
from jax.experimental.pallas import tpu as pltpu
from jax.experimental.pallas import tpu_sc as plsc

N = 10000
E = 320000
D = 128
NC = 2
NS = 16
NW = NC * NS
C = 128
CH_W = 80
CH0 = 120
CH1 = 40
E_PAD = NW * CH_W * C
EW = CH_W * C
N_PAD = 10112
ROWS_T = N_PAD // NS
HR = 128
HRU = 80
BM = 400


def _sc_mesh():
    return plsc.VectorSubcoreMesh(core_axis_name="c", subcore_axis_name="s")


def _deg_body(cols_hbm, iota_hbm, zeros_hbm, deg_out,
              idx_v, iota_v, hist, deg_sh):
    c = lax.axis_index("c")
    s = lax.axis_index("s")
    w = c * NS + s
    pltpu.sync_copy(zeros_hbm.at[pl.ds(0, HR // NS)],
                    deg_sh.at[pl.ds(s * (HR // NS), HR // NS)])
    pltpu.sync_copy(zeros_hbm, hist)
    pltpu.sync_copy(iota_hbm, iota_v)
    pltpu.sync_copy(cols_hbm.at[w], idx_v)
    plsc.subcore_barrier()

    ones16 = jnp.ones((16,), jnp.float32)

    def step(i, carry):
        v = idx_v[pl.ds(i * 16, 16)]
        plsc.addupdate_scatter(hist, [v >> 7, v & 127], ones16)
        return carry

    lax.fori_loop(0, EW // 16, step, 0)
    pltpu.sync_copy(hist.at[pl.ds(0, HRU)], deg_sh.at[iota_v], add=True)
    plsc.subcore_barrier()

    rt = HR // NS
    pltpu.sync_copy(deg_sh.at[pl.ds(s * rt, rt)],
                    deg_out.at[c, pl.ds(s * rt, rt)])


def _deg_kernel(cols_r, iota, zeros128):
    f = pl.kernel(
        _deg_body,
        out_type=jax.ShapeDtypeStruct((NC, HR, 128), jnp.float32),
        mesh=_sc_mesh(),
        scratch_types=(
            pltpu.VMEM((EW,), jnp.int32),
            pltpu.VMEM((HRU,), jnp.int32),
            pltpu.VMEM((HR, 128), jnp.float32),
            pltpu.VMEM_SHARED((HR, 128), jnp.float32),
        ),
        compiler_params=pltpu.CompilerParams(needs_layout_passes=False),
    )
    return f(cols_r, iota, zeros128)


def _mm_body(x_ref, w_ref, d0_ref, d1_ref, y_ref):
    xw = jnp.dot(x_ref[...], w_ref[...], preferred_element_type=jnp.float32)
    deg = d0_ref[...] + d1_ref[...] + 1.0
    y_ref[...] = xw * lax.rsqrt(deg)


def _mm(x, W, deg0, deg1):
    return pl.pallas_call(
        _mm_body,
        grid=(N // BM,),
        in_specs=[
            pl.BlockSpec((BM, D), lambda i: (i, 0)),
            pl.BlockSpec((D, D), lambda i: (0, 0)),
            pl.BlockSpec((BM, 1), lambda i: (i, 0)),
            pl.BlockSpec((BM, 1), lambda i: (i, 0)),
        ],
        out_specs=pl.BlockSpec((BM, D), lambda i: (i, 0)),
        out_shape=jax.ShapeDtypeStruct((N, D), jnp.float32),
    )(x, W, deg0, deg1)


def _acc_body(y_hbm, rows_hbm, cols_hbm, zeros_hbm, acc_out,
              rv0, rv1, cols_v, buf0, buf1, g0, g1, i0, i1, acc_sh):
    c = lax.axis_index("c")
    s = lax.axis_index("s")
    pltpu.sync_copy(zeros_hbm, acc_sh.at[pl.ds(s * ROWS_T, ROWS_T)])
    ch = jnp.where(c == 0, CH0, CH1)
    base = c * (NS * CH0) + s * ch

    @pl.when(c == 0)
    def _():
        pltpu.sync_copy(cols_hbm.at[pl.ds(s * CH0, CH0)],
                        cols_v.at[pl.ds(0, CH0)])

    @pl.when(c == 1)
    def _():
        pltpu.sync_copy(cols_hbm.at[pl.ds(NS * CH0 + s * CH1, CH1)],
                        cols_v.at[pl.ds(0, CH1)])

    plsc.subcore_barrier()

    pltpu.async_copy(rows_hbm.at[base], rv0, i0)
    pltpu.async_copy(rows_hbm.at[base + 1], rv1, i1)
    pltpu.make_async_copy(rows_hbm.at[base], rv0, i0).wait()
    pltpu.async_copy(y_hbm.at[rv0], buf0, g0)
    pltpu.make_async_copy(rows_hbm.at[base + 1], rv1, i1).wait()
    pltpu.async_copy(y_hbm.at[rv1], buf1, g1)

    def step(t, carry):
        j0 = 2 * t
        j1 = j0 + 1
        pltpu.make_async_copy(y_hbm.at[rv0], buf0, g0).wait()

        @pl.when(j0 + 2 < ch)
        def _():
            pltpu.async_copy(rows_hbm.at[base + j0 + 2], rv0, i0)

        pltpu.sync_copy(buf0, acc_sh.at[cols_v.at[j0]], add=True)

        @pl.when(j0 + 2 < ch)
        def _():
            pltpu.make_async_copy(rows_hbm.at[base + j0 + 2], rv0, i0).wait()
            pltpu.async_copy(y_hbm.at[rv0], buf0, g0)

        pltpu.make_async_copy(y_hbm.at[rv1], buf1, g1).wait()

        @pl.when(j1 + 2 < ch)
        def _():
            pltpu.async_copy(rows_hbm.at[base + j1 + 2], rv1, i1)

        pltpu.sync_copy(buf1, acc_sh.at[cols_v.at[j1]], add=True)

        @pl.when(j1 + 2 < ch)
        def _():
            pltpu.make_async_copy(rows_hbm.at[base + j1 + 2], rv1, i1).wait()
            pltpu.async_copy(y_hbm.at[rv1], buf1, g1)

        return carry

    lax.fori_loop(0, ch // 2, step, 0)
    plsc.subcore_barrier()

    sl = pl.ds(s * ROWS_T, ROWS_T)
    pltpu.sync_copy(acc_sh.at[sl], acc_out.at[c, sl])


def _acc_kernel(y, rows_r, cols_r, zeros):
    f = pl.kernel(
        _acc_body,
        out_type=jax.ShapeDtypeStruct((NC, N_PAD, D), jnp.float32),
        mesh=_sc_mesh(),
        scratch_types=(
            pltpu.VMEM((C,), jnp.int32),
            pltpu.VMEM((C,), jnp.int32),
            pltpu.VMEM((CH0, C), jnp.int32),
            pltpu.VMEM((C, D), jnp.float32),
            pltpu.VMEM((C, D), jnp.float32),
            pltpu.SemaphoreType.DMA,
            pltpu.SemaphoreType.DMA,
            pltpu.SemaphoreType.DMA,
            pltpu.SemaphoreType.DMA,
            pltpu.VMEM_SHARED((N_PAD, D), jnp.float32),
        ),
    )
    return f(y, rows_r, cols_r, zeros)


def _fin_body(a0_ref, a1_ref, y_ref, d0_ref, d1_ref, b_ref, o_ref):
    deg = d0_ref[...] + d1_ref[...] + 1.0
    o_ref[...] = lax.rsqrt(deg) * (a0_ref[...] + a1_ref[...] + y_ref[...]) + b_ref[...]


def _combine(acc0, acc1, y, deg0, deg1, b2d):
    return pl.pallas_call(
        _fin_body,
        grid=(N // BM,),
        in_specs=[
            pl.BlockSpec((BM, D), lambda i: (i, 0)),
            pl.BlockSpec((BM, D), lambda i: (i, 0)),
            pl.BlockSpec((BM, D), lambda i: (i, 0)),
            pl.BlockSpec((BM, 1), lambda i: (i, 0)),
            pl.BlockSpec((BM, 1), lambda i: (i, 0)),
            pl.BlockSpec((1, D), lambda i: (0, 0)),
        ],
        out_specs=pl.BlockSpec((BM, D), lambda i: (i, 0)),
        out_shape=jax.ShapeDtypeStruct((N, D), jnp.float32),
    )(acc0, acc1, y, deg0, deg1, b2d)


def kernel(mpg_ft, edge_index, W, b):
    ei = edge_index.astype(jnp.int32)
    pad = E_PAD - E
    rows_p = jnp.concatenate([ei[0], jnp.zeros((pad,), jnp.int32)])
    cols_p = jnp.concatenate(
        [ei[1], N + (jnp.arange(pad, dtype=jnp.int32) % (N_PAD - N))])
    rows_r = rows_p.reshape(E_PAD // C, C)
    cols_r = cols_p.reshape(E_PAD // C, C)

    iota = jnp.arange(HRU, dtype=jnp.int32)
    zeros128 = jnp.zeros((HR, 128), jnp.float32)
    zeros = jnp.zeros((ROWS_T, D), jnp.float32)

    degs = _deg_kernel(cols_p.reshape(NW, EW), iota, zeros128)
    deg0 = degs[0].reshape(HR * 128, 1)[:N]
    deg1 = degs[1].reshape(HR * 128, 1)[:N]
    y = _mm(mpg_ft, W, deg0, deg1)
    accs = _acc_kernel(y, rows_r, cols_r, zeros)
    return _combine(accs[0, :N], accs[1, :N], y, deg0, deg1, b.reshape(1, D))


# speedup vs baseline: 16.5730x; 1.0010x over previous
import jax
import jax.numpy as jnp
from jax import lax
from jax.experimental import pallas as pl
from jax.experimental.pallas import tpu as pltpu
from jax.experimental.pallas import tpu_sc as plsc

N = 10000
E = 320000
D = 128
NC = 2
NS = 16
NW = NC * NS
C = 128
CH0 = 120
CH1 = 40
E_PAD = NS * (CH0 + CH1) * C
EW = E_PAD // NW
N_PAD = 10112
ROWS_T = N_PAD // NS
HR = 128
HRU = 80
BM = 400


def _sc_mesh():
    return plsc.VectorSubcoreMesh(core_axis_name="c", subcore_axis_name="s")


def _deg_body(cols_hbm, iota_hbm, zeros_hbm, deg_out,
              idx_v, iota_v, hist, deg_sh):
    c = lax.axis_index("c")
    s = lax.axis_index("s")
    w = c * NS + s
    pltpu.sync_copy(zeros_hbm.at[pl.ds(0, HR // NS)],
                    deg_sh.at[pl.ds(s * (HR // NS), HR // NS)])
    pltpu.sync_copy(zeros_hbm, hist)
    pltpu.sync_copy(iota_hbm, iota_v)
    pltpu.sync_copy(cols_hbm.at[w], idx_v)
    plsc.subcore_barrier()

    ones16 = jnp.ones((16,), jnp.float32)

    def step(i, carry):
        v = idx_v[pl.ds(i * 16, 16)]
        plsc.addupdate_scatter(hist, [v >> 7, v & 127], ones16)
        return carry

    lax.fori_loop(0, EW // 16, step, 0)
    pltpu.sync_copy(hist.at[pl.ds(0, HRU)], deg_sh.at[iota_v], add=True)
    plsc.subcore_barrier()

    rt = HR // NS
    pltpu.sync_copy(deg_sh.at[pl.ds(s * rt, rt)],
                    deg_out.at[c, pl.ds(s * rt, rt)])


def _deg_kernel(cols_r, iota, zeros128):
    f = pl.kernel(
        _deg_body,
        out_type=jax.ShapeDtypeStruct((NC, HR, 128), jnp.float32),
        mesh=_sc_mesh(),
        scratch_types=(
            pltpu.VMEM((EW,), jnp.int32),
            pltpu.VMEM((HRU,), jnp.int32),
            pltpu.VMEM((HR, 128), jnp.float32),
            pltpu.VMEM_SHARED((HR, 128), jnp.float32),
        ),
        compiler_params=pltpu.CompilerParams(needs_layout_passes=False),
    )
    return f(cols_r, iota, zeros128)


def _mm_body(x_ref, w_ref, d0_ref, d1_ref, y_ref):
    xw = jnp.dot(x_ref[...], w_ref[...], preferred_element_type=jnp.float32)
    deg = d0_ref[...] + d1_ref[...] + 1.0
    y_ref[...] = xw * lax.rsqrt(deg)


def _mm(x, W, deg0, deg1):
    return pl.pallas_call(
        _mm_body,
        grid=(N // BM,),
        in_specs=[
            pl.BlockSpec((BM, D), lambda i: (i, 0)),
            pl.BlockSpec((D, D), lambda i: (0, 0)),
            pl.BlockSpec((BM, 1), lambda i: (i, 0)),
            pl.BlockSpec((BM, 1), lambda i: (i, 0)),
        ],
        out_specs=pl.BlockSpec((BM, D), lambda i: (i, 0)),
        out_shape=jax.ShapeDtypeStruct((N, D), jnp.float32),
    )(x, W, deg0, deg1)


def _acc_body(y_hbm, rows_hbm, cols_hbm, zeros_hbm, acc_out,
              rows_v, cv0, cv1, cv2, cv3, buf0, buf1,
              g0, g1, ic0, ic1, ic2, ic3, acc_sh):
    c = lax.axis_index("c")
    s = lax.axis_index("s")
    pltpu.sync_copy(zeros_hbm, acc_sh.at[pl.ds(s * ROWS_T, ROWS_T)])
    ch = jnp.where(c == 0, CH0, CH1)
    base = c * (NS * CH0) + s * ch

    @pl.when(c == 0)
    def _():
        pltpu.sync_copy(rows_hbm.at[pl.ds(s * CH0, CH0)],
                        rows_v.at[pl.ds(0, CH0)])

    @pl.when(c == 1)
    def _():
        pltpu.sync_copy(rows_hbm.at[pl.ds(NS * CH0 + s * CH1, CH1)],
                        rows_v.at[pl.ds(0, CH1)])

    plsc.subcore_barrier()

    cvs = (cv0, cv1, cv2, cv3)
    ics = (ic0, ic1, ic2, ic3)
    bufs = (buf0, buf1)
    gs = (g0, g1)
    for v in range(4):
        pltpu.async_copy(cols_hbm.at[base + v], cvs[v], ics[v])
    pltpu.async_copy(y_hbm.at[rows_v.at[0]], buf0, g0)
    pltpu.async_copy(y_hbm.at[rows_v.at[1]], buf1, g1)

    def step(u, carry):
        for v in range(4):
            j = 4 * u + v
            b = v % 2
            pltpu.make_async_copy(y_hbm.at[rows_v.at[j]], bufs[b], gs[b]).wait()
            pltpu.make_async_copy(cols_hbm.at[base + j], cvs[v], ics[v]).wait()
            pltpu.sync_copy(bufs[b], acc_sh.at[cvs[v]], add=True)

            @pl.when(j + 4 < ch)
            def _():
                pltpu.async_copy(cols_hbm.at[base + j + 4], cvs[v], ics[v])

            @pl.when(j + 2 < ch)
            def _():
                pltpu.async_copy(y_hbm.at[rows_v.at[j + 2]], bufs[b], gs[b])

        return carry

    lax.fori_loop(0, ch // 4, step, 0)
    plsc.subcore_barrier()

    sl = pl.ds(s * ROWS_T, ROWS_T)
    pltpu.sync_copy(acc_sh.at[sl], acc_out.at[c, sl])


def _acc_kernel(y, rows_r, cols_r, zeros):
    f = pl.kernel(
        _acc_body,
        out_type=jax.ShapeDtypeStruct((NC, N_PAD, D), jnp.float32),
        mesh=_sc_mesh(),
        scratch_types=(
            pltpu.VMEM((CH0, C), jnp.int32),
            pltpu.VMEM((C,), jnp.int32),
            pltpu.VMEM((C,), jnp.int32),
            pltpu.VMEM((C,), jnp.int32),
            pltpu.VMEM((C,), jnp.int32),
            pltpu.VMEM((C, D), jnp.float32),
            pltpu.VMEM((C, D), jnp.float32),
            pltpu.SemaphoreType.DMA,
            pltpu.SemaphoreType.DMA,
            pltpu.SemaphoreType.DMA,
            pltpu.SemaphoreType.DMA,
            pltpu.SemaphoreType.DMA,
            pltpu.SemaphoreType.DMA,
            pltpu.VMEM_SHARED((N_PAD, D), jnp.float32),
        ),
    )
    return f(y, rows_r, cols_r, zeros)


def _fin_body(a0_ref, a1_ref, y_ref, d0_ref, d1_ref, b_ref, o_ref):
    deg = d0_ref[...] + d1_ref[...] + 1.0
    o_ref[...] = lax.rsqrt(deg) * (a0_ref[...] + a1_ref[...] + y_ref[...]) + b_ref[...]


def _combine(acc0, acc1, y, deg0, deg1, b2d):
    return pl.pallas_call(
        _fin_body,
        grid=(N // BM,),
        in_specs=[
            pl.BlockSpec((BM, D), lambda i: (i, 0)),
            pl.BlockSpec((BM, D), lambda i: (i, 0)),
            pl.BlockSpec((BM, D), lambda i: (i, 0)),
            pl.BlockSpec((BM, 1), lambda i: (i, 0)),
            pl.BlockSpec((BM, 1), lambda i: (i, 0)),
            pl.BlockSpec((1, D), lambda i: (0, 0)),
        ],
        out_specs=pl.BlockSpec((BM, D), lambda i: (i, 0)),
        out_shape=jax.ShapeDtypeStruct((N, D), jnp.float32),
    )(acc0, acc1, y, deg0, deg1, b2d)


def kernel(mpg_ft, edge_index, W, b):
    ei = edge_index.astype(jnp.int32)
    pad = E_PAD - E
    rows_p = jnp.concatenate([ei[0], jnp.zeros((pad,), jnp.int32)])
    cols_p = jnp.concatenate(
        [ei[1], N + (jnp.arange(pad, dtype=jnp.int32) % (N_PAD - N))])
    rows_r = rows_p.reshape(E_PAD // C, C)
    cols_r = cols_p.reshape(E_PAD // C, C)

    iota = jnp.arange(HRU, dtype=jnp.int32)
    zeros128 = jnp.zeros((HR, 128), jnp.float32)
    zeros = jnp.zeros((ROWS_T, D), jnp.float32)

    degs = _deg_kernel(cols_p.reshape(NW, EW), iota, zeros128)
    deg0 = degs[0].reshape(HR * 128, 1)[:N]
    deg1 = degs[1].reshape(HR * 128, 1)[:N]
    y = _mm(mpg_ft, W, deg0, deg1)
    accs = _acc_kernel(y, rows_r, cols_r, zeros)
    return _combine(accs[0, :N], accs[1, :N], y, deg0, deg1, b.reshape(1, D))
